# baseline (speedup 1.0000x reference)
import jax
import jax.numpy as jnp
from jax.experimental import pallas as pl
from jax.experimental.pallas import tpu as pltpu

_P = 20000
_N = 32
_NX, _NY, _NZ = 216, 248, 1
_B = 4
_C = 64
_PX, _PY, _PZ = 0.32, 0.32, 4.0
_XOFF = _PX / 2 + 0.0
_YOFF = _PY / 2 + (-39.68)
_ZOFF = _PZ / 2 + (-3.0)

_TP = 400
_NT = _P // _TP
_NCELLS = _NZ * _NY * _NX
_TOT = _B * _NCELLS
_NBLK = 8
_CB = _TOT // _NBLK


def _augment(pillars, coords_f, npts):
    xyz = pillars[:, :, :3]
    xyz_mean = jnp.sum(xyz, axis=1, keepdims=True) / npts[:, :, None]
    cluster = xyz - xyz_mean
    cx = xyz[:, :, 0] - (coords_f[:, 3][:, None] * _PX + _XOFF)
    cy = xyz[:, :, 1] - (coords_f[:, 2][:, None] * _PY + _YOFF)
    cz = xyz[:, :, 2] - (coords_f[:, 1][:, None] * _PZ + _ZOFF)
    center = jnp.stack([cx, cy, cz], axis=-1)
    return jnp.concatenate([pillars, cluster, center], axis=-1)


def _stats_kernel(pillars_ref, coords_ref, npts_ref, wt_ref, out_ref):
    feat = _augment(pillars_ref[...], coords_ref[...], npts_ref[...])
    h = jnp.dot(feat.reshape(_TP * _N, 10), wt_ref[...],
                preferred_element_type=jnp.float32)
    part = jnp.stack([jnp.sum(h, axis=0), jnp.sum(h * h, axis=0)])

    @pl.when(pl.program_id(0) == 0)
    def _init():
        out_ref[...] = part

    @pl.when(pl.program_id(0) != 0)
    def _acc():
        out_ref[...] += part


def _pv_kernel(pillars_ref, coords_ref, npts_ref, wts_ref, shift_ref, out_ref):
    feat = _augment(pillars_ref[...], coords_ref[...], npts_ref[...])
    h = jnp.dot(feat.reshape(_TP * _N, 10), wts_ref[...],
                preferred_element_type=jnp.float32) + shift_ref[...]
    h = jnp.maximum(h, 0.0)
    out_ref[...] = jnp.max(h.reshape(_TP, _N, _C), axis=1)


def _scatter_kernel(flat_ref, pv_ref, out_ref):
    base = pl.program_id(0) * _CB
    out_ref[...] = jnp.zeros((_CB, _C), jnp.float32)

    def body(p, carry):
        local = flat_ref[p] - base

        @pl.when((local >= 0) & (local < _CB))
        def _store():
            out_ref[pl.ds(local, 1), :] = pv_ref[pl.ds(p, 1), :]

        return carry

    jax.lax.fori_loop(0, _P, body, 0)


@jax.jit
def kernel(pillars, coords, num_points, W, gamma, beta):
    pillars = pillars.astype(jnp.float32)
    coords = coords.astype(jnp.int32)
    coords_f = coords.astype(jnp.float32)
    npts = jnp.maximum(num_points, 1).astype(jnp.float32).reshape(_P, 1)
    wt = W.astype(jnp.float32).T

    tile_specs = [
        pl.BlockSpec((_TP, _N, 4), lambda i: (i, 0, 0)),
        pl.BlockSpec((_TP, 4), lambda i: (i, 0)),
        pl.BlockSpec((_TP, 1), lambda i: (i, 0)),
    ]

    sums = pl.pallas_call(
        _stats_kernel,
        grid=(_NT,),
        in_specs=tile_specs + [pl.BlockSpec((10, _C), lambda i: (0, 0))],
        out_specs=pl.BlockSpec((2, _C), lambda i: (0, 0)),
        out_shape=jax.ShapeDtypeStruct((2, _C), jnp.float32),
    )(pillars, coords_f, npts, wt)

    n = float(_P * _N)
    mean = sums[0] / n
    var = sums[1] / n - mean * mean
    scale = gamma.astype(jnp.float32) / jnp.sqrt(var + 1e-3)
    wts = wt * scale[None, :]
    shift = (beta.astype(jnp.float32) - mean * scale).reshape(1, _C)

    pv = pl.pallas_call(
        _pv_kernel,
        grid=(_NT,),
        in_specs=tile_specs + [
            pl.BlockSpec((10, _C), lambda i: (0, 0)),
            pl.BlockSpec((1, _C), lambda i: (0, 0)),
        ],
        out_specs=pl.BlockSpec((_TP, _C), lambda i: (i, 0)),
        out_shape=jax.ShapeDtypeStruct((_P, _C), jnp.float32),
    )(pillars, coords_f, npts, wts, shift)

    flat = coords[:, 0] * _NCELLS + coords[:, 1] + coords[:, 2] * _NX + coords[:, 3]

    canvas = pl.pallas_call(
        _scatter_kernel,
        grid_spec=pltpu.PrefetchScalarGridSpec(
            num_scalar_prefetch=1,
            grid=(_NBLK,),
            in_specs=[pl.BlockSpec((_P, _C), lambda i, flat_ref: (0, 0))],
            out_specs=pl.BlockSpec((_CB, _C), lambda i, flat_ref: (i, 0)),
        ),
        out_shape=jax.ShapeDtypeStruct((_TOT, _C), jnp.float32),
    )(flat, pv)

    fm = canvas.reshape(_B, _NCELLS, _C).transpose(0, 2, 1)
    return fm.reshape(_B, _C * _NZ, _NY, _NX)


# speedup vs baseline: 2.7954x; 2.7954x over previous
import jax
import jax.numpy as jnp
from jax.experimental import pallas as pl
from jax.experimental.pallas import tpu as pltpu

_P = 20000
_N = 32
_NX, _NY, _NZ = 216, 248, 1
_B = 4
_C = 64
_PX, _PY, _PZ = 0.32, 0.32, 4.0
_XOFF = _PX / 2 + 0.0
_YOFF = _PY / 2 + (-39.68)
_ZOFF = _PZ / 2 + (-3.0)

_TP = 400
_NT = _P // _TP
_NCELLS = _NZ * _NY * _NX
_TOT = _B * _NCELLS
_NBLK = 8
_CB = _TOT // _NBLK


def _augment(pillars, coords_f, npts):
    xyz = pillars[:, :, :3]
    xyz_mean = jnp.sum(xyz, axis=1, keepdims=True) / npts[:, :, None]
    cluster = xyz - xyz_mean
    cx = xyz[:, :, 0] - (coords_f[:, 3][:, None] * _PX + _XOFF)
    cy = xyz[:, :, 1] - (coords_f[:, 2][:, None] * _PY + _YOFF)
    cz = xyz[:, :, 2] - (coords_f[:, 1][:, None] * _PZ + _ZOFF)
    center = jnp.stack([cx, cy, cz], axis=-1)
    return jnp.concatenate([pillars, cluster, center], axis=-1)


def _stats_kernel(pillars_ref, coords_ref, npts_ref, wt_ref, out_ref):
    feat = _augment(pillars_ref[...], coords_ref[...], npts_ref[...])
    h = jnp.dot(feat.reshape(_TP * _N, 10), wt_ref[...],
                preferred_element_type=jnp.float32)
    part = jnp.stack([jnp.sum(h, axis=0), jnp.sum(h * h, axis=0)])

    @pl.when(pl.program_id(0) == 0)
    def _init():
        out_ref[...] = part

    @pl.when(pl.program_id(0) != 0)
    def _acc():
        out_ref[...] += part


def _pv_kernel(pillars_ref, coords_ref, npts_ref, wts_ref, shift_ref, out_ref):
    feat = _augment(pillars_ref[...], coords_ref[...], npts_ref[...])
    h = jnp.dot(feat.reshape(_TP * _N, 10), wts_ref[...],
                preferred_element_type=jnp.float32) + shift_ref[...]
    h = jnp.maximum(h, 0.0)
    out_ref[...] = jnp.max(h.reshape(_TP, _N, _C), axis=1)


def _scatter_kernel(sflat_ref, order_ref, starts_ref, pv_ref, out_ref):
    blk = pl.program_id(0)
    base = blk * _CB
    out_ref[...] = jnp.zeros((_CB, _C), jnp.float32)

    def body(i, carry):
        local = sflat_ref[i] - base
        p = order_ref[i]
        out_ref[pl.ds(local, 1), :] = pv_ref[pl.ds(p, 1), :]
        return carry

    jax.lax.fori_loop(starts_ref[blk], starts_ref[blk + 1], body, 0)


@jax.jit
def kernel(pillars, coords, num_points, W, gamma, beta):
    pillars = pillars.astype(jnp.float32)
    coords = coords.astype(jnp.int32)
    coords_f = coords.astype(jnp.float32)
    npts = jnp.maximum(num_points, 1).astype(jnp.float32).reshape(_P, 1)
    wt = W.astype(jnp.float32).T

    tile_specs = [
        pl.BlockSpec((_TP, _N, 4), lambda i: (i, 0, 0)),
        pl.BlockSpec((_TP, 4), lambda i: (i, 0)),
        pl.BlockSpec((_TP, 1), lambda i: (i, 0)),
    ]

    sums = pl.pallas_call(
        _stats_kernel,
        grid=(_NT,),
        in_specs=tile_specs + [pl.BlockSpec((10, _C), lambda i: (0, 0))],
        out_specs=pl.BlockSpec((2, _C), lambda i: (0, 0)),
        out_shape=jax.ShapeDtypeStruct((2, _C), jnp.float32),
    )(pillars, coords_f, npts, wt)

    n = float(_P * _N)
    mean = sums[0] / n
    var = sums[1] / n - mean * mean
    scale = gamma.astype(jnp.float32) / jnp.sqrt(var + 1e-3)
    wts = wt * scale[None, :]
    shift = (beta.astype(jnp.float32) - mean * scale).reshape(1, _C)

    pv = pl.pallas_call(
        _pv_kernel,
        grid=(_NT,),
        in_specs=tile_specs + [
            pl.BlockSpec((10, _C), lambda i: (0, 0)),
            pl.BlockSpec((1, _C), lambda i: (0, 0)),
        ],
        out_specs=pl.BlockSpec((_TP, _C), lambda i: (i, 0)),
        out_shape=jax.ShapeDtypeStruct((_P, _C), jnp.float32),
    )(pillars, coords_f, npts, wts, shift)

    flat = coords[:, 0] * _NCELLS + coords[:, 1] + coords[:, 2] * _NX + coords[:, 3]
    order = jnp.argsort(flat, stable=True).astype(jnp.int32)
    sflat = flat[order]
    starts = jnp.searchsorted(
        sflat, jnp.arange(_NBLK + 1, dtype=jnp.int32) * _CB
    ).astype(jnp.int32)

    canvas = pl.pallas_call(
        _scatter_kernel,
        grid_spec=pltpu.PrefetchScalarGridSpec(
            num_scalar_prefetch=3,
            grid=(_NBLK,),
            in_specs=[pl.BlockSpec((_P, _C), lambda i, *_: (0, 0))],
            out_specs=pl.BlockSpec((_CB, _C), lambda i, *_: (i, 0)),
        ),
        out_shape=jax.ShapeDtypeStruct((_TOT, _C), jnp.float32),
    )(sflat, order, starts, pv)

    fm = canvas.reshape(_B, _NCELLS, _C).transpose(0, 2, 1)
    return fm.reshape(_B, _C * _NZ, _NY, _NX)
